# baseline (speedup 1.0000x reference)
import functools

import jax
import jax.numpy as jnp
from jax import lax
from jax.experimental import pallas as pl
from jax.experimental.pallas import tpu as pltpu
from jax.experimental.pallas import tpu_sc as plsc

N = 10000
H = 256
VIS = 128
RB = 400
GRID_R = N // RB
CB = 2000
GRID_C = N // CB

_f32 = jnp.float32


def _bspec(shape, imap):
    return pl.BlockSpec(shape, imap)


def _mm(xs, ws, bias, affine=None, relu=True, stats=True):
    nx = len(xs)
    M = ws[0].shape[1]

    def body(*refs):
        xrefs = refs[:nx]
        wrefs = refs[nx:2 * nx]
        idx = 2 * nx
        bref = refs[idx]; idx += 1
        if affine is not None:
            saref = refs[idx]; caref = refs[idx + 1]; idx += 2
        oref = refs[idx]; idx += 1
        stref = refs[idx] if stats else None

        x0 = xrefs[0][...]
        if affine is not None:
            x0 = x0 * saref[...] + caref[...]
        acc = jnp.dot(x0, wrefs[0][...], preferred_element_type=_f32)
        for k in range(1, nx):
            acc = acc + jnp.dot(xrefs[k][...], wrefs[k][...],
                                preferred_element_type=_f32)
        acc = acc + bref[...]
        if relu:
            acc = jnp.maximum(acc, 0.0)
        oref[...] = acc
        if stats:
            i = pl.program_id(0)
            upd = jnp.concatenate(
                [jnp.sum(acc, 0)[None], jnp.sum(acc * acc, 0)[None],
                 jnp.zeros((6, M), _f32)], 0)

            @pl.when(i == 0)
            def _():
                stref[...] = upd

            @pl.when(i > 0)
            def _():
                stref[...] = stref[...] + upd

    in_specs = [_bspec((RB, x.shape[1]), lambda i: (i, 0)) for x in xs]
    in_specs += [_bspec(w.shape, lambda i: (0, 0)) for w in ws]
    in_specs += [_bspec((1, M), lambda i: (0, 0))]
    inputs = list(xs) + list(ws) + [bias.reshape(1, M)]
    if affine is not None:
        K0 = xs[0].shape[1]
        in_specs += [_bspec((1, K0), lambda i: (0, 0))] * 2
        inputs += [affine[0].reshape(1, K0), affine[1].reshape(1, K0)]
    out_shape = [jax.ShapeDtypeStruct((N, M), _f32)]
    out_specs = [_bspec((RB, M), lambda i: (i, 0))]
    if stats:
        out_shape.append(jax.ShapeDtypeStruct((8, M), _f32))
        out_specs.append(_bspec((8, M), lambda i: (0, 0)))
    res = pl.pallas_call(
        body, grid=(GRID_R,), in_specs=in_specs, out_specs=out_specs,
        out_shape=out_shape,
        compiler_params=pltpu.CompilerParams(
            dimension_semantics=("arbitrary",)),
    )(*inputs)
    return res if stats else res[0]


def _bn_aff(sums, g, be):
    m = sums[0] / N
    v = sums[1] / N - m * m
    s = g * jax.lax.rsqrt(v + 1e-5)
    return s, be - m * s


def _hst(fx, W, a2):
    def body(fxref, wref, aref, href, stref):
        h = jnp.dot(fxref[...], wref[...], preferred_element_type=_f32)
        href[...] = h
        stref[...] = jnp.dot(h, aref[...], preferred_element_type=_f32)

    return pl.pallas_call(
        body, grid=(GRID_R,),
        in_specs=[_bspec((RB, H), lambda i: (i, 0)),
                  _bspec((H, H), lambda i: (0, 0)),
                  _bspec((H, 2), lambda i: (0, 0))],
        out_specs=[_bspec((RB, H), lambda i: (i, 0)),
                   _bspec((RB, 2), lambda i: (i, 0))],
        out_shape=[jax.ShapeDtypeStruct((N, H), _f32),
                   jax.ShapeDtypeStruct((N, 2), _f32)],
        compiler_params=pltpu.CompilerParams(
            dimension_semantics=("arbitrary",)),
    )(fx, W, a2)


def _gat_dense(adj, st, h):
    GB = 200

    def body(adjref, strref, stcref, href, hpref, rsref):
        s = strref[:, 0]
        t = stcref[:, 1]
        p1 = jnp.exp(-s)[:, None] * jnp.exp(-t)[None, :]
        p2 = jnp.exp(-0.1 * s)[:, None] * jnp.exp(-0.1 * t)[None, :]
        E = adjref[...] * jnp.minimum(p1, p2)
        hpref[...] = jnp.dot(E.astype(jnp.bfloat16),
                             href[...].astype(jnp.bfloat16),
                             preferred_element_type=_f32)
        rsref[...] = jnp.sum(E, axis=1, keepdims=True)

    return pl.pallas_call(
        body, grid=(N // GB,),
        in_specs=[_bspec((GB, N), lambda i: (i, 0)),
                  _bspec((GB, 2), lambda i: (i, 0)),
                  _bspec((N, 2), lambda i: (0, 0)),
                  _bspec((N, H), lambda i: (0, 0))],
        out_specs=[_bspec((GB, H), lambda i: (i, 0)),
                   _bspec((GB, 1), lambda i: (i, 0))],
        out_shape=[jax.ShapeDtypeStruct((N, H), _f32),
                   jax.ShapeDtypeStruct((N, 1), _f32)],
        compiler_params=pltpu.CompilerParams(
            dimension_semantics=("arbitrary",)),
    )(adj, st, st, h)


def _post_agg(hp, rs):
    def body(hpref, rsref, rref, stref):
        i = pl.program_id(0)
        r = jnp.maximum(hpref[...], 0.0) / (rsref[...] + 1e-5)
        rref[...] = r
        upd = jnp.concatenate(
            [jnp.sum(r, 0)[None], jnp.sum(r * r, 0)[None],
             jnp.zeros((6, H), _f32)], 0)

        @pl.when(i == 0)
        def _():
            stref[...] = upd

        @pl.when(i > 0)
        def _():
            stref[...] = stref[...] + upd

    return pl.pallas_call(
        body, grid=(GRID_R,),
        in_specs=[_bspec((RB, H), lambda i: (i, 0)),
                  _bspec((RB, 1), lambda i: (i, 0))],
        out_specs=[_bspec((RB, H), lambda i: (i, 0)),
                   _bspec((8, H), lambda i: (0, 0))],
        out_shape=[jax.ShapeDtypeStruct((N, H), _f32),
                   jax.ShapeDtypeStruct((8, H), _f32)],
        compiler_params=pltpu.CompilerParams(
            dimension_semantics=("arbitrary",)),
    )(hp, rs)


def _residual(r, fx, sbn, cbn):
    def body(rref, fxref, sref, cref, oref):
        oref[...] = rref[...] * sref[...] + cref[...] + fxref[...]

    return pl.pallas_call(
        body, grid=(GRID_R,),
        in_specs=[_bspec((RB, H), lambda i: (i, 0)),
                  _bspec((RB, H), lambda i: (i, 0)),
                  _bspec((1, H), lambda i: (0, 0)),
                  _bspec((1, H), lambda i: (0, 0))],
        out_specs=_bspec((RB, H), lambda i: (i, 0)),
        out_shape=jax.ShapeDtypeStruct((N, H), _f32),
        compiler_params=pltpu.CompilerParams(
            dimension_semantics=("arbitrary",)),
    )(r, fx, sbn.reshape(1, H), cbn.reshape(1, H))


NGRP = 640
_i32 = jnp.int32
NP = 10240
_TROWS = NP // 32


def _bitpack(adj):
    GB = 200

    def body(adjref, outref):
        r = lax.broadcasted_iota(_i32, (2000, 128), 0)
        g = lax.broadcasted_iota(_i32, (2000, 128), 1)
        w = jnp.where((r // 16) == g,
                      lax.shift_left(jnp.int32(1), r % 16), 0).astype(_f32)
        pieces = []
        for c in range(5):
            blk = adjref[:, 2000 * c:2000 * (c + 1)]
            m = jnp.dot(blk, w, preferred_element_type=_f32)
            pieces.append(m[:, :125].astype(_i32))
        outref[...] = jnp.concatenate(
            pieces + [jnp.zeros((GB, NGRP - 625), _i32)], axis=1)

    return pl.pallas_call(
        body, grid=(N // GB,),
        in_specs=[_bspec((GB, N), lambda i: (i, 0))],
        out_specs=_bspec((GB, NGRP), lambda i: (i, 0)),
        out_shape=jax.ShapeDtypeStruct((N, NGRP), _i32),
        compiler_params=pltpu.CompilerParams(
            dimension_semantics=("arbitrary",)),
    )(adj)


def _sc_mesh():
    return plsc.VectorSubcoreMesh(core_axis_name="c", subcore_axis_name="s",
                                  num_cores=2, num_subcores=16)


def _sc_extract(bm):
    def body(bm_hbm, nbr_hbm, deg_hbm, b0, b1, degs_v, gids_v, gmasks_v,
             nbrbuf_v, sem0, sem1):
        lane = lax.iota(_i32, 16)
        wid = lax.axis_index("s") * 2 + lax.axis_index("c")
        base = wid * _TROWS

        def decode(mrow, u_local):
            cg = jnp.zeros((16,), _i32)
            for c in range(NGRP // 16):
                grp = mrow[pl.ds(16 * c, 16)]
                nz = grp != 0
                gid = jnp.full((16,), 16 * c, _i32) + lane
                pos = cg + plsc.cumsum(jnp.where(nz, 1, 0)) - 1
                plsc.store_scatter(gids_v, [pos], gid, mask=nz)
                plsc.store_scatter(gmasks_v, [pos], grp, mask=nz)
                cg = cg + plsc.all_reduce_population_count(nz)
            nbrbuf_v[pl.ds(0, 16)] = jnp.zeros((16,), _i32)
            nbrbuf_v[pl.ds(16, 16)] = jnp.zeros((16,), _i32)
            plsc.subcore_barrier()
            gidvec = gids_v[pl.ds(0, 16)]
            mskvec = gmasks_v[pl.ds(0, 16)]
            ce = jnp.zeros((16,), _i32)
            for j in range(16):
                jj = jnp.full((16,), j, _i32)
                gid = gidvec[jj]
                msk = mskvec[jj]
                bits = lax.shift_right_logical(msk, lane) & 1
                sel = (bits == 1) & (jj < cg)
                cols = gid * 16 + lane
                pos = ce + plsc.cumsum(jnp.where(sel, 1, 0)) - 1
                plsc.store_scatter(nbrbuf_v, [pos], cols, mask=sel)
                ce = ce + plsc.all_reduce_population_count(sel)
            plsc.store_scatter(degs_v, [jnp.full((16,), u_local, _i32)],
                               ce, mask=lane == 0)
            plsc.subcore_barrier()
            pltpu.sync_copy(nbrbuf_v.at[pl.ds(0, 16)],
                            nbr_hbm.at[pl.ds((base + u_local) * 16, 16)])

        def bmrow(i):
            return bm_hbm.at[pl.ds(jnp.minimum(base + i, NP - 1) * NGRP, NGRP)]

        pltpu.async_copy(bmrow(0), b0, sem0)

        def pair(i, _):
            u = 2 * i
            pltpu.make_async_copy(bmrow(0), b0, sem0).wait()
            pltpu.async_copy(bmrow(u + 1), b1, sem1)
            decode(b0, u)
            pltpu.make_async_copy(bmrow(0), b1, sem1).wait()
            pltpu.async_copy(bmrow(u + 2), b0, sem0)
            decode(b1, u + 1)
            return 0

        lax.fori_loop(0, _TROWS // 2, pair, 0)
        pltpu.make_async_copy(bmrow(0), b0, sem0).wait()
        pltpu.sync_copy(degs_v, deg_hbm.at[pl.ds(base, _TROWS)])

    f = pl.kernel(
        body,
        out_type=[jax.ShapeDtypeStruct((NP * 16,), _i32),
                  jax.ShapeDtypeStruct((NP,), _i32)],
        mesh=_sc_mesh(),
        compiler_params=pltpu.CompilerParams(needs_layout_passes=False),
        scratch_types=[pltpu.VMEM((NGRP,), _i32), pltpu.VMEM((NGRP,), _i32),
                       pltpu.VMEM((_TROWS,), _i32),
                       pltpu.VMEM((32,), _i32), pltpu.VMEM((32,), _i32),
                       pltpu.VMEM((32,), _i32),
                       pltpu.SemaphoreType.DMA, pltpu.SemaphoreType.DMA],
    )
    return f(bm)


def _sc_agg(nbr, deg, h, s, t):
    lane = lax.iota(_i32, 16)

    def body(nbr_hbm, deg_hbm, h_hbm, s_hbm, t_hbm, hp_hbm, rs_hbm,
             nbr_v, deg_v, s_v, t_v, rs_v, acc_v, hb0, hb1,
             sem0, sem1):
        lane = lax.iota(_i32, 16)
        wid = lax.axis_index("s") * 2 + lax.axis_index("c")
        base = wid * _TROWS
        pltpu.sync_copy(nbr_hbm.at[pl.ds(base * 16, _TROWS * 16)], nbr_v)
        pltpu.sync_copy(deg_hbm.at[pl.ds(base, _TROWS)], deg_v)
        pltpu.sync_copy(s_hbm.at[pl.ds(base, _TROWS)], s_v)
        pltpu.sync_copy(t_hbm, t_v)

        def gather(u_local, hb, sem):
            uc = jnp.minimum(u_local, _TROWS - 1)
            idx = plsc.load_gather(nbr_v, [jnp.full((16,), uc * 16, _i32)
                                           + lane])
            pltpu.async_copy(h_hbm.at[idx], hb, sem)
            return idx

        def compute(u_local, idx, hb):
            uu = jnp.full((16,), u_local, _i32)
            su = plsc.load_gather(s_v, [uu])
            du = plsc.load_gather(deg_v, [uu])
            tg = plsc.load_gather(t_v, [idx])
            z = su + tg
            e = jnp.exp(-jnp.where(z > 0, z, 0.1 * z))
            e = jnp.where(lane < du, e, 0.0)
            rs = jnp.sum(e)
            plsc.store_scatter(rs_v, [uu], jnp.full((16,), rs, _f32),
                               mask=lane == 0)
            accs = [jnp.zeros((16,), _f32) for _ in range(16)]
            for j in range(16):
                ej = e[jnp.full((16,), j, _i32)]
                for c in range(16):
                    accs[c] = accs[c] + ej * hb[j, pl.ds(16 * c, 16)]
            for c in range(16):
                acc_v[pl.ds(16 * c, 16)] = accs[c]
            pltpu.sync_copy(acc_v, hp_hbm.at[pl.ds((base + u_local) * H, H)])

        i0 = gather(0, hb0, sem0)

        def pair(i, idx0):
            u = 2 * i
            pltpu.make_async_copy(h_hbm.at[idx0], hb0, sem0).wait()
            idx1 = gather(u + 1, hb1, sem1)
            compute(u, idx0, hb0)
            pltpu.make_async_copy(h_hbm.at[idx1], hb1, sem1).wait()
            idx2 = gather(u + 2, hb0, sem0)
            compute(u + 1, idx1, hb1)
            return idx2

        idlast = lax.fori_loop(0, _TROWS // 2, pair, i0)
        pltpu.make_async_copy(h_hbm.at[idlast], hb0, sem0).wait()
        pltpu.sync_copy(rs_v, rs_hbm.at[pl.ds(base, _TROWS)])

    f = pl.kernel(
        body,
        out_type=[jax.ShapeDtypeStruct((NP * H,), _f32),
                  jax.ShapeDtypeStruct((NP,), _f32)],
        mesh=_sc_mesh(),
        compiler_params=pltpu.CompilerParams(needs_layout_passes=False),
        scratch_types=[pltpu.VMEM((_TROWS * 16,), _i32),
                       pltpu.VMEM((_TROWS,), _i32),
                       pltpu.VMEM((_TROWS,), _f32),
                       pltpu.VMEM((NP,), _f32),
                       pltpu.VMEM((_TROWS,), _f32),
                       pltpu.VMEM((H,), _f32),
                       pltpu.VMEM((16, H), _f32), pltpu.VMEM((16, H), _f32),
                       pltpu.SemaphoreType.DMA, pltpu.SemaphoreType.DMA],
    )
    return f(nbr, deg, h, s, t)


def _vl_head(o0, o1, o2, goal, info, v):
    w0 = v['w0']
    w0s = [w0[:H], w0[H:2 * H], w0[2 * H:3 * H],
           w0[3 * H:3 * H + VIS], w0[3 * H + VIS:]]

    def body(o0r, o1r, o2r, gr, ir, w0a, w0b, w0c, w0d, w0e,
             b0r, w1r, b1r, w2r, b2r, w3r, b3r, outr):
        _bf = jnp.bfloat16

        def bdot(a, b):
            return jnp.dot(a.astype(_bf), b.astype(_bf),
                           preferred_element_type=_f32)

        x = bdot(o0r[...], w0a[...])
        x += bdot(o1r[...], w0b[...])
        x += bdot(o2r[...], w0c[...])
        x += bdot(gr[...], w0d[...])
        x += bdot(ir[...], w0e[...])
        x = jnp.maximum(x + b0r[...], 0.0)
        x = jnp.maximum(bdot(x, w1r[...]) + b1r[...], 0.0)
        x = jnp.maximum(bdot(x, w2r[...]) + b2r[...], 0.0)
        x = bdot(x, w3r[...]) + b3r[...]
        outr[...] = 1.0 / (1.0 + jnp.exp(-x))

    H2, H4 = 2 * H, 4 * H
    in_specs = [_bspec((RB, H), lambda i: (i, 0))] * 3
    in_specs += [_bspec((RB, VIS), lambda i: (i, 0)),
                 _bspec((RB, 4), lambda i: (i, 0))]
    in_specs += [_bspec(w.shape, lambda i: (0, 0)) for w in w0s]
    in_specs += [_bspec((1, H2), lambda i: (0, 0)),
                 _bspec((H2, H2), lambda i: (0, 0)),
                 _bspec((1, H2), lambda i: (0, 0)),
                 _bspec((H2, H4), lambda i: (0, 0)),
                 _bspec((1, H4), lambda i: (0, 0)),
                 _bspec((H4, 1), lambda i: (0, 0)),
                 _bspec((1, 1), lambda i: (0, 0))]
    return pl.pallas_call(
        body, grid=(GRID_R,),
        in_specs=in_specs,
        out_specs=_bspec((RB, 1), lambda i: (i, 0)),
        out_shape=jax.ShapeDtypeStruct((N, 1), _f32),
        compiler_params=pltpu.CompilerParams(
            dimension_semantics=("arbitrary",)),
    )(o0, o1, o2, goal, info, *w0s,
      v['b0'].reshape(1, H2), v['w1'], v['b1'].reshape(1, H2),
      v['w2'], v['b2'].reshape(1, H4), v['w3'], v['b3'].reshape(1, 1))


def _mlp3(xs, ws0, q):
    y1, s1 = _mm(xs, ws0, q['b0'])
    af1 = _bn_aff(s1, q['g0'], q['be0'])
    y2, s2 = _mm([y1], [q['w1']], q['b1'], affine=af1)
    af2 = _bn_aff(s2, q['g1'], q['be1'])
    return _mm([y2], [q['w2']], q['b2'], affine=af2, relu=False, stats=False)


def kernel(feat, goal_feat, info_feat, adj, params):
    p = params
    fx0 = _mlp3([feat], [p['fe']['w0']], p['fe'])

    def ne(fx, q):
        return _mlp3([fx, goal_feat, info_feat],
                     [q['w0'][:H], q['w0'][H:H + VIS], q['w0'][H + VIS:]], q)

    def a2(a):
        return jnp.stack([a[0, :H], a[0, H:]], axis=1)

    bm = jnp.pad(_bitpack(adj), ((0, NP - N), (0, 0))).reshape(-1)
    nbr, deg = _sc_extract(bm)

    def gat_layer(fx, W, a):
        h, st = _hst(fx, W, a2(a))
        sp = jnp.pad(st[:, 0], (0, NP - N))
        tp = jnp.pad(st[:, 1], (0, NP - N))
        hp, rs = _sc_agg(nbr, deg, h, sp, tp)
        return _post_agg(hp.reshape(NP, H)[:N], rs[:N].reshape(N, 1))

    out0 = ne(fx0, p['ne0'])
    r0, sr0 = gat_layer(fx0, p['ga0_W'], p['ga0_a'])
    bs0, bc0 = _bn_aff(sr0, p['bn_g'], p['bn_b'])
    fx1 = _residual(r0, fx0, bs0, bc0)

    out1 = ne(fx1, p['ne1'])
    r1, sr1 = gat_layer(fx1, p['ga1_W'], p['ga1_a'])
    bs1, bc1 = _bn_aff(sr1, p['bn_g'], p['bn_b'])
    fx2 = _residual(r1, fx1, bs1, bc1)

    out2 = ne(fx2, p['ne2'])
    return _vl_head(out0, out1, out2, goal_feat, info_feat, p['vl'])


# speedup vs baseline: 1.1817x; 1.1817x over previous
import functools

import jax
import jax.numpy as jnp
from jax import lax
from jax.experimental import pallas as pl
from jax.experimental.pallas import tpu as pltpu
from jax.experimental.pallas import tpu_sc as plsc

N = 10000
H = 256
VIS = 128
RB = 400
GRID_R = N // RB
CB = 2000
GRID_C = N // CB

_f32 = jnp.float32


def _bspec(shape, imap):
    return pl.BlockSpec(shape, imap)


def _mm(xs, ws, bias, affine=None, relu=True, stats=True):
    nx = len(xs)
    M = ws[0].shape[1]

    def body(*refs):
        xrefs = refs[:nx]
        wrefs = refs[nx:2 * nx]
        idx = 2 * nx
        bref = refs[idx]; idx += 1
        if affine is not None:
            saref = refs[idx]; caref = refs[idx + 1]; idx += 2
        oref = refs[idx]; idx += 1
        stref = refs[idx] if stats else None

        x0 = xrefs[0][...]
        if affine is not None:
            x0 = x0 * saref[...] + caref[...]
        acc = jnp.dot(x0, wrefs[0][...], preferred_element_type=_f32)
        for k in range(1, nx):
            acc = acc + jnp.dot(xrefs[k][...], wrefs[k][...],
                                preferred_element_type=_f32)
        acc = acc + bref[...]
        if relu:
            acc = jnp.maximum(acc, 0.0)
        oref[...] = acc
        if stats:
            i = pl.program_id(0)
            upd = jnp.concatenate(
                [jnp.sum(acc, 0)[None], jnp.sum(acc * acc, 0)[None],
                 jnp.zeros((6, M), _f32)], 0)

            @pl.when(i == 0)
            def _():
                stref[...] = upd

            @pl.when(i > 0)
            def _():
                stref[...] = stref[...] + upd

    in_specs = [_bspec((RB, x.shape[1]), lambda i: (i, 0)) for x in xs]
    in_specs += [_bspec(w.shape, lambda i: (0, 0)) for w in ws]
    in_specs += [_bspec((1, M), lambda i: (0, 0))]
    inputs = list(xs) + list(ws) + [bias.reshape(1, M)]
    if affine is not None:
        K0 = xs[0].shape[1]
        in_specs += [_bspec((1, K0), lambda i: (0, 0))] * 2
        inputs += [affine[0].reshape(1, K0), affine[1].reshape(1, K0)]
    out_shape = [jax.ShapeDtypeStruct((N, M), _f32)]
    out_specs = [_bspec((RB, M), lambda i: (i, 0))]
    if stats:
        out_shape.append(jax.ShapeDtypeStruct((8, M), _f32))
        out_specs.append(_bspec((8, M), lambda i: (0, 0)))
    res = pl.pallas_call(
        body, grid=(GRID_R,), in_specs=in_specs, out_specs=out_specs,
        out_shape=out_shape,
        compiler_params=pltpu.CompilerParams(
            dimension_semantics=("arbitrary",)),
    )(*inputs)
    return res if stats else res[0]


def _bn_aff(sums, g, be):
    m = sums[0] / N
    v = sums[1] / N - m * m
    s = g * jax.lax.rsqrt(v + 1e-5)
    return s, be - m * s


def _hst(fx, W, a2):
    def body(fxref, wref, aref, href, stref):
        h = jnp.dot(fxref[...], wref[...], preferred_element_type=_f32)
        href[...] = h
        stref[...] = jnp.dot(h, aref[...], preferred_element_type=_f32)

    return pl.pallas_call(
        body, grid=(GRID_R,),
        in_specs=[_bspec((RB, H), lambda i: (i, 0)),
                  _bspec((H, H), lambda i: (0, 0)),
                  _bspec((H, 2), lambda i: (0, 0))],
        out_specs=[_bspec((RB, H), lambda i: (i, 0)),
                   _bspec((RB, 2), lambda i: (i, 0))],
        out_shape=[jax.ShapeDtypeStruct((N, H), _f32),
                   jax.ShapeDtypeStruct((N, 2), _f32)],
        compiler_params=pltpu.CompilerParams(
            dimension_semantics=("arbitrary",)),
    )(fx, W, a2)


def _gat_dense(adj, st, h):
    GB = 200

    def body(adjref, strref, stcref, href, hpref, rsref):
        s = strref[:, 0]
        t = stcref[:, 1]
        p1 = jnp.exp(-s)[:, None] * jnp.exp(-t)[None, :]
        p2 = jnp.exp(-0.1 * s)[:, None] * jnp.exp(-0.1 * t)[None, :]
        E = adjref[...] * jnp.minimum(p1, p2)
        hpref[...] = jnp.dot(E.astype(jnp.bfloat16),
                             href[...].astype(jnp.bfloat16),
                             preferred_element_type=_f32)
        rsref[...] = jnp.sum(E, axis=1, keepdims=True)

    return pl.pallas_call(
        body, grid=(N // GB,),
        in_specs=[_bspec((GB, N), lambda i: (i, 0)),
                  _bspec((GB, 2), lambda i: (i, 0)),
                  _bspec((N, 2), lambda i: (0, 0)),
                  _bspec((N, H), lambda i: (0, 0))],
        out_specs=[_bspec((GB, H), lambda i: (i, 0)),
                   _bspec((GB, 1), lambda i: (i, 0))],
        out_shape=[jax.ShapeDtypeStruct((N, H), _f32),
                   jax.ShapeDtypeStruct((N, 1), _f32)],
        compiler_params=pltpu.CompilerParams(
            dimension_semantics=("arbitrary",)),
    )(adj, st, st, h)


def _post_agg(hp, rs):
    def body(hpref, rsref, rref, stref):
        i = pl.program_id(0)
        r = jnp.maximum(hpref[...], 0.0) / (rsref[...] + 1e-5)
        rref[...] = r
        upd = jnp.concatenate(
            [jnp.sum(r, 0)[None], jnp.sum(r * r, 0)[None],
             jnp.zeros((6, H), _f32)], 0)

        @pl.when(i == 0)
        def _():
            stref[...] = upd

        @pl.when(i > 0)
        def _():
            stref[...] = stref[...] + upd

    return pl.pallas_call(
        body, grid=(GRID_R,),
        in_specs=[_bspec((RB, H), lambda i: (i, 0)),
                  _bspec((RB, 1), lambda i: (i, 0))],
        out_specs=[_bspec((RB, H), lambda i: (i, 0)),
                   _bspec((8, H), lambda i: (0, 0))],
        out_shape=[jax.ShapeDtypeStruct((N, H), _f32),
                   jax.ShapeDtypeStruct((8, H), _f32)],
        compiler_params=pltpu.CompilerParams(
            dimension_semantics=("arbitrary",)),
    )(hp, rs)


def _residual(r, fx, sbn, cbn):
    def body(rref, fxref, sref, cref, oref):
        oref[...] = rref[...] * sref[...] + cref[...] + fxref[...]

    return pl.pallas_call(
        body, grid=(GRID_R,),
        in_specs=[_bspec((RB, H), lambda i: (i, 0)),
                  _bspec((RB, H), lambda i: (i, 0)),
                  _bspec((1, H), lambda i: (0, 0)),
                  _bspec((1, H), lambda i: (0, 0))],
        out_specs=_bspec((RB, H), lambda i: (i, 0)),
        out_shape=jax.ShapeDtypeStruct((N, H), _f32),
        compiler_params=pltpu.CompilerParams(
            dimension_semantics=("arbitrary",)),
    )(r, fx, sbn.reshape(1, H), cbn.reshape(1, H))


NGRP = 640
_i32 = jnp.int32
NP = 10240
_TROWS = NP // 32


def _bitpack(adj):
    GB = 200

    def body(adjref, outref):
        r = lax.broadcasted_iota(_i32, (2000, 128), 0)
        g = lax.broadcasted_iota(_i32, (2000, 128), 1)
        w = jnp.where((r // 16) == g,
                      lax.shift_left(jnp.int32(1), r % 16), 0).astype(_f32)
        pieces = []
        for c in range(5):
            blk = adjref[:, 2000 * c:2000 * (c + 1)]
            m = jnp.dot(blk, w, preferred_element_type=_f32)
            pieces.append(m[:, :125].astype(_i32))
        outref[...] = jnp.concatenate(
            pieces + [jnp.zeros((GB, NGRP - 625), _i32)], axis=1)

    return pl.pallas_call(
        body, grid=(N // GB,),
        in_specs=[_bspec((GB, N), lambda i: (i, 0))],
        out_specs=_bspec((GB, NGRP), lambda i: (i, 0)),
        out_shape=jax.ShapeDtypeStruct((N, NGRP), _i32),
        compiler_params=pltpu.CompilerParams(
            dimension_semantics=("arbitrary",)),
    )(adj)


def _sc_mesh():
    return plsc.VectorSubcoreMesh(core_axis_name="c", subcore_axis_name="s",
                                  num_cores=2, num_subcores=16)


def _sc_extract(bm):
    def body(bm_hbm, nbr_hbm, deg_hbm, b0, b1, g0, m0, g1, m1,
             degs_v, nbr_v, sem0, sem1):
        lane = lax.iota(_i32, 16)
        wid = lax.axis_index("s") * 2 + lax.axis_index("c")
        base = wid * _TROWS

        def zrow(i, _):
            plsc.store_scatter(nbr_v, [i * 16 + lane],
                               jnp.zeros((16,), _i32))
            return 0

        lax.fori_loop(0, _TROWS, zrow, 0)

        def pass1(mrow, gv, mv):
            cg = jnp.zeros((16,), _i32)
            for c in range(NGRP // 16):
                grp = mrow[pl.ds(16 * c, 16)]
                nz = grp != 0
                gid = jnp.full((16,), 16 * c, _i32) + lane
                pos = cg + plsc.cumsum(jnp.where(nz, 1, 0)) - 1
                plsc.store_scatter(gv, [pos], gid, mask=nz)
                plsc.store_scatter(mv, [pos], grp, mask=nz)
                cg = cg + plsc.all_reduce_population_count(nz)
            return cg

        def pass2(u_local, gv, mv, cg):
            uu = jnp.full((16,), u_local, _i32)
            gidvec = gv[pl.ds(0, 16)]
            mskvec = mv[pl.ds(0, 16)]
            ce = jnp.zeros((16,), _i32)
            for j in range(16):
                jj = jnp.full((16,), j, _i32)
                gid = gidvec[jj]
                msk = mskvec[jj]
                bits = lax.shift_right_logical(msk, lane) & 1
                sel = (bits == 1) & (jj < cg)
                cols = gid * 16 + lane
                pos = ce + plsc.cumsum(jnp.where(sel, 1, 0)) - 1
                plsc.store_scatter(nbr_v, [uu * 16 + pos], cols, mask=sel)
                ce = ce + plsc.all_reduce_population_count(sel)
            plsc.store_scatter(degs_v, [uu], ce, mask=lane == 0)

        def bmrow(i):
            return bm_hbm.at[pl.ds(jnp.minimum(base + i, NP - 1) * NGRP, NGRP)]

        pltpu.async_copy(bmrow(0), b0, sem0)
        pltpu.async_copy(bmrow(1), b1, sem1)
        pltpu.make_async_copy(bmrow(0), b0, sem0).wait()
        cg0_init = pass1(b0, g0, m0)
        pltpu.async_copy(bmrow(2), b0, sem0)

        def pair(i, cg0):
            u = 2 * i
            pltpu.make_async_copy(bmrow(0), b1, sem1).wait()
            cg1 = pass1(b1, g1, m1)
            pltpu.async_copy(bmrow(u + 3), b1, sem1)
            pass2(u, g0, m0, cg0)
            pltpu.make_async_copy(bmrow(0), b0, sem0).wait()
            cg0n = pass1(b0, g0, m0)
            pltpu.async_copy(bmrow(u + 4), b0, sem0)
            pass2(u + 1, g1, m1, cg1)
            return cg0n

        lax.fori_loop(0, _TROWS // 2, pair, cg0_init)
        pltpu.make_async_copy(bmrow(0), b0, sem0).wait()
        pltpu.make_async_copy(bmrow(0), b1, sem1).wait()
        plsc.subcore_barrier()
        pltpu.sync_copy(nbr_v, nbr_hbm.at[pl.ds(base * 16, _TROWS * 16)])
        pltpu.sync_copy(degs_v, deg_hbm.at[pl.ds(base, _TROWS)])

    f = pl.kernel(
        body,
        out_type=[jax.ShapeDtypeStruct((NP * 16,), _i32),
                  jax.ShapeDtypeStruct((NP,), _i32)],
        mesh=_sc_mesh(),
        compiler_params=pltpu.CompilerParams(needs_layout_passes=False),
        scratch_types=[pltpu.VMEM((NGRP,), _i32), pltpu.VMEM((NGRP,), _i32),
                       pltpu.VMEM((32,), _i32), pltpu.VMEM((32,), _i32),
                       pltpu.VMEM((32,), _i32), pltpu.VMEM((32,), _i32),
                       pltpu.VMEM((_TROWS,), _i32),
                       pltpu.VMEM((_TROWS * 16,), _i32),
                       pltpu.SemaphoreType.DMA, pltpu.SemaphoreType.DMA],
    )
    return f(bm)


def _sc_agg(nbr, deg, h, s, t):
    def body(nbr_hbm, deg_hbm, h_hbm, s_hbm, t_hbm, hp_hbm, rs_hbm,
             nbr_v, deg_v, s_v, t_v, rs_v, hp_v,
             hb0, hb1, hb2, hb3, sem0, sem1, sem2, sem3):
        lane = lax.iota(_i32, 16)
        wid = lax.axis_index("s") * 2 + lax.axis_index("c")
        base = wid * _TROWS
        pltpu.sync_copy(nbr_hbm.at[pl.ds(base * 16, _TROWS * 16)], nbr_v)
        pltpu.sync_copy(deg_hbm.at[pl.ds(base, _TROWS)], deg_v)
        pltpu.sync_copy(s_hbm.at[pl.ds(base, _TROWS)], s_v)
        pltpu.sync_copy(t_hbm, t_v)
        hbs = (hb0, hb1, hb2, hb3)
        sems = (sem0, sem1, sem2, sem3)

        def gather(u_local, hb, sem):
            uc = jnp.minimum(u_local, _TROWS - 1)
            idx = plsc.load_gather(nbr_v, [jnp.full((16,), uc * 16, _i32)
                                           + lane])
            pltpu.async_copy(h_hbm.at[idx], hb, sem)
            return idx

        def compute(u_local, hb):
            uu = jnp.full((16,), u_local, _i32)
            su = plsc.load_gather(s_v, [uu])
            du = plsc.load_gather(deg_v, [uu])
            idx = plsc.load_gather(nbr_v, [uu * 16 + lane])
            tg = plsc.load_gather(t_v, [idx])
            z = su + tg
            e = jnp.exp(-jnp.where(z > 0, z, 0.1 * z))
            e = jnp.where(lane < du, e, 0.0)
            rs = jnp.sum(e)
            plsc.store_scatter(rs_v, [uu], jnp.full((16,), rs, _f32),
                               mask=lane == 0)
            for c in range(16):
                acc = jnp.zeros((16,), _f32)
                for j in range(16):
                    ej = e[jnp.full((16,), j, _i32)]
                    acc = acc + ej * hb[j, pl.ds(16 * c, 16)]
                plsc.store_scatter(hp_v, [uu * H + 16 * c + lane], acc)

        idxs = [gather(k, hbs[k], sems[k]) for k in range(4)]

        def quad(i, carry):
            i0, i1, i2, i3 = carry
            u = 4 * i
            outs = []
            for k, ik in enumerate((i0, i1, i2, i3)):
                pltpu.make_async_copy(h_hbm.at[ik], hbs[k], sems[k]).wait()
                compute(u + k, hbs[k])
                outs.append(gather(u + k + 4, hbs[k], sems[k]))
            return tuple(outs)

        last = lax.fori_loop(0, _TROWS // 4, quad, tuple(idxs))
        for k in range(4):
            pltpu.make_async_copy(h_hbm.at[last[k]], hbs[k], sems[k]).wait()
        plsc.subcore_barrier()
        pltpu.sync_copy(hp_v, hp_hbm.at[pl.ds(base * H, _TROWS * H)])
        pltpu.sync_copy(rs_v, rs_hbm.at[pl.ds(base, _TROWS)])

    f = pl.kernel(
        body,
        out_type=[jax.ShapeDtypeStruct((NP * H,), _f32),
                  jax.ShapeDtypeStruct((NP,), _f32)],
        mesh=_sc_mesh(),
        compiler_params=pltpu.CompilerParams(needs_layout_passes=False),
        scratch_types=[pltpu.VMEM((_TROWS * 16,), _i32),
                       pltpu.VMEM((_TROWS,), _i32),
                       pltpu.VMEM((_TROWS,), _f32),
                       pltpu.VMEM((NP,), _f32),
                       pltpu.VMEM((_TROWS,), _f32),
                       pltpu.VMEM((_TROWS * H,), _f32),
                       pltpu.VMEM((16, H), _f32), pltpu.VMEM((16, H), _f32),
                       pltpu.VMEM((16, H), _f32), pltpu.VMEM((16, H), _f32),
                       pltpu.SemaphoreType.DMA, pltpu.SemaphoreType.DMA,
                       pltpu.SemaphoreType.DMA, pltpu.SemaphoreType.DMA],
    )
    return f(nbr, deg, h, s, t)


def _vl_head(o0, o1, o2, goal, info, v):
    w0 = v['w0']
    w0s = [w0[:H], w0[H:2 * H], w0[2 * H:3 * H],
           w0[3 * H:3 * H + VIS], w0[3 * H + VIS:]]

    def body(o0r, o1r, o2r, gr, ir, w0a, w0b, w0c, w0d, w0e,
             b0r, w1r, b1r, w2r, b2r, w3r, b3r, outr):
        _bf = jnp.bfloat16

        def bdot(a, b):
            return jnp.dot(a.astype(_bf), b.astype(_bf),
                           preferred_element_type=_f32)

        x = bdot(o0r[...], w0a[...])
        x += bdot(o1r[...], w0b[...])
        x += bdot(o2r[...], w0c[...])
        x += bdot(gr[...], w0d[...])
        x += bdot(ir[...], w0e[...])
        x = jnp.maximum(x + b0r[...], 0.0)
        x = jnp.maximum(bdot(x, w1r[...]) + b1r[...], 0.0)
        x = jnp.maximum(bdot(x, w2r[...]) + b2r[...], 0.0)
        x = bdot(x, w3r[...]) + b3r[...]
        outr[...] = 1.0 / (1.0 + jnp.exp(-x))

    H2, H4 = 2 * H, 4 * H
    in_specs = [_bspec((RB, H), lambda i: (i, 0))] * 3
    in_specs += [_bspec((RB, VIS), lambda i: (i, 0)),
                 _bspec((RB, 4), lambda i: (i, 0))]
    in_specs += [_bspec(w.shape, lambda i: (0, 0)) for w in w0s]
    in_specs += [_bspec((1, H2), lambda i: (0, 0)),
                 _bspec((H2, H2), lambda i: (0, 0)),
                 _bspec((1, H2), lambda i: (0, 0)),
                 _bspec((H2, H4), lambda i: (0, 0)),
                 _bspec((1, H4), lambda i: (0, 0)),
                 _bspec((H4, 1), lambda i: (0, 0)),
                 _bspec((1, 1), lambda i: (0, 0))]
    return pl.pallas_call(
        body, grid=(GRID_R,),
        in_specs=in_specs,
        out_specs=_bspec((RB, 1), lambda i: (i, 0)),
        out_shape=jax.ShapeDtypeStruct((N, 1), _f32),
        compiler_params=pltpu.CompilerParams(
            dimension_semantics=("arbitrary",)),
    )(o0, o1, o2, goal, info, *w0s,
      v['b0'].reshape(1, H2), v['w1'], v['b1'].reshape(1, H2),
      v['w2'], v['b2'].reshape(1, H4), v['w3'], v['b3'].reshape(1, 1))


def _mlp3(xs, ws0, q):
    y1, s1 = _mm(xs, ws0, q['b0'])
    af1 = _bn_aff(s1, q['g0'], q['be0'])
    y2, s2 = _mm([y1], [q['w1']], q['b1'], affine=af1)
    af2 = _bn_aff(s2, q['g1'], q['be1'])
    return _mm([y2], [q['w2']], q['b2'], affine=af2, relu=False, stats=False)


def kernel(feat, goal_feat, info_feat, adj, params):
    p = params
    fx0 = _mlp3([feat], [p['fe']['w0']], p['fe'])

    def ne(fx, q):
        return _mlp3([fx, goal_feat, info_feat],
                     [q['w0'][:H], q['w0'][H:H + VIS], q['w0'][H + VIS:]], q)

    def a2(a):
        return jnp.stack([a[0, :H], a[0, H:]], axis=1)

    bm = jnp.pad(_bitpack(adj), ((0, NP - N), (0, 0))).reshape(-1)
    nbr, deg = _sc_extract(bm)

    def gat_layer(fx, W, a):
        h, st = _hst(fx, W, a2(a))
        sp = jnp.pad(st[:, 0], (0, NP - N))
        tp = jnp.pad(st[:, 1], (0, NP - N))
        hp, rs = _sc_agg(nbr, deg, h, sp, tp)
        return _post_agg(hp.reshape(NP, H)[:N], rs[:N].reshape(N, 1))

    out0 = ne(fx0, p['ne0'])
    r0, sr0 = gat_layer(fx0, p['ga0_W'], p['ga0_a'])
    bs0, bc0 = _bn_aff(sr0, p['bn_g'], p['bn_b'])
    fx1 = _residual(r0, fx0, bs0, bc0)

    out1 = ne(fx1, p['ne1'])
    r1, sr1 = gat_layer(fx1, p['ga1_W'], p['ga1_a'])
    bs1, bc1 = _bn_aff(sr1, p['bn_g'], p['bn_b'])
    fx2 = _residual(r1, fx1, bs1, bc1)

    out2 = ne(fx2, p['ne2'])
    return _vl_head(out0, out1, out2, goal_feat, info_feat, p['vl'])
